# Initial kernel scaffold; baseline (speedup 1.0000x reference)
import jax
import jax.numpy as jnp
from jax.experimental import pallas as pl


def kernel(x, edge_index, W0, b0, g0, beta0, W1, b1, W2, b2, W3, b3):
    raise NotImplementedError("write your pallas kernel here")


# speedup vs baseline: 6.1957x; 6.1957x over previous
import functools

import jax
import jax.numpy as jnp
from jax import lax
from jax.experimental import pallas as pl
from jax.experimental.pallas import tpu as pltpu
from jax.experimental.pallas import tpu_sc as plsc

N = 10000
E = 320000
H = 128

NC = 2
NS = 16
NW = NC * NS
NPAD = 10240
RPS = NPAD // NS
EPW = E // NW
K = 80
CHUNKS = EPW // K

_mesh = plsc.VectorSubcoreMesh(core_axis_name="c", subcore_axis_name="s")


@functools.partial(
    pl.kernel,
    out_type=(
        jax.ShapeDtypeStruct((NC, NPAD, 16), jnp.float32),
        jax.ShapeDtypeStruct((NC, NPAD, 16), jnp.float32),
    ),
    mesh=_mesh,
    scratch_types=[
        pltpu.VMEM((K,), jnp.int32),
        pltpu.VMEM((K,), jnp.int32),
        pltpu.VMEM((K, 16), jnp.float32),
        pltpu.VMEM_SHARED((NPAD, 16), jnp.float32),
        pltpu.VMEM_SHARED((NPAD, 16), jnp.float32),
    ],
)
def _sc_degree(src_hbm, dst_hbm, zeros_hbm, ones_hbm,
               outs_hbm, outd_hbm,
               si_v, di_v, ones_v, degs_sh, degd_sh):
    c = lax.axis_index("c")
    s = lax.axis_index("s")
    wid = s * NC + c
    r0 = s * RPS
    pltpu.sync_copy(ones_hbm, ones_v)
    pltpu.sync_copy(zeros_hbm.at[pl.ds(r0, RPS)], degs_sh.at[pl.ds(r0, RPS)])
    pltpu.sync_copy(zeros_hbm.at[pl.ds(r0, RPS)], degd_sh.at[pl.ds(r0, RPS)])
    plsc.subcore_barrier()

    def body(j, carry):
        base = wid * EPW + j * K
        pltpu.sync_copy(src_hbm.at[pl.ds(base, K)], si_v)
        pltpu.sync_copy(dst_hbm.at[pl.ds(base, K)], di_v)
        pltpu.sync_copy(ones_v, degs_sh.at[si_v], add=True)
        pltpu.sync_copy(ones_v, degd_sh.at[di_v], add=True)
        return carry

    lax.fori_loop(0, CHUNKS, body, 0)
    plsc.subcore_barrier()
    pltpu.sync_copy(degs_sh.at[pl.ds(r0, RPS)], outs_hbm.at[c, pl.ds(r0, RPS)])
    pltpu.sync_copy(degd_sh.at[pl.ds(r0, RPS)], outd_hbm.at[c, pl.ds(r0, RPS)])


@functools.partial(
    pl.kernel,
    out_type=jax.ShapeDtypeStruct((NC, NPAD, H), jnp.float32),
    mesh=_mesh,
    scratch_types=[
        pltpu.VMEM((K,), jnp.int32),
        pltpu.VMEM((K,), jnp.int32),
        pltpu.VMEM((K, H), jnp.float32),
        pltpu.VMEM_SHARED((NPAD, H), jnp.float32),
        pltpu.SemaphoreType.DMA,
    ],
)
def _sc_aggregate(hs_hbm, src_hbm, dst_hbm, zeros_hbm, out_hbm,
                  si_v, di_v, rows_v, agg_sh, sem):
    c = lax.axis_index("c")
    s = lax.axis_index("s")
    wid = s * NC + c
    r0 = s * RPS
    pltpu.sync_copy(zeros_hbm.at[pl.ds(r0, RPS)], agg_sh.at[pl.ds(r0, RPS)])
    plsc.subcore_barrier()

    def body(j, carry):
        base = wid * EPW + j * K
        pltpu.sync_copy(src_hbm.at[pl.ds(base, K)], si_v)
        pltpu.sync_copy(dst_hbm.at[pl.ds(base, K)], di_v)
        pltpu.async_copy(hs_hbm.at[si_v], rows_v, sem).wait()
        pltpu.sync_copy(rows_v, agg_sh.at[di_v], add=True)
        return carry

    lax.fori_loop(0, CHUNKS, body, 0)
    plsc.subcore_barrier()
    pltpu.sync_copy(agg_sh.at[pl.ds(r0, RPS)], out_hbm.at[c, pl.ds(r0, RPS)])


BR = 1000


def _tc_mlp_body(x_ref, w_ref, b_ref, g_ref, bt_ref, degs_ref,
                 h_ref, hs_ref):
    t = jnp.dot(x_ref[...], w_ref[...], preferred_element_type=jnp.float32)
    t = t + b_ref[...]
    mu = jnp.mean(t, axis=-1, keepdims=True)
    var = jnp.mean((t - mu) * (t - mu), axis=-1, keepdims=True)
    t = (t - mu) * lax.rsqrt(var + 1e-5) * g_ref[...] + bt_ref[...]
    h = jnp.maximum(t, 0.0)
    ds_ = degs_ref[0, :, 0:1] + degs_ref[1, :, 0:1] + 1.0
    h_ref[...] = h
    hs_ref[...] = h * lax.rsqrt(ds_)


def _tc_mlp(x, W0, b0, g0, beta0, degs):
    return pl.pallas_call(
        _tc_mlp_body,
        grid=(N // BR,),
        in_specs=[
            pl.BlockSpec((BR, H), lambda i: (i, 0)),
            pl.BlockSpec((H, H), lambda i: (0, 0)),
            pl.BlockSpec((1, H), lambda i: (0, 0)),
            pl.BlockSpec((1, H), lambda i: (0, 0)),
            pl.BlockSpec((1, H), lambda i: (0, 0)),
            pl.BlockSpec((NC, BR, 16), lambda i: (0, i, 0)),
        ],
        out_specs=[
            pl.BlockSpec((BR, H), lambda i: (i, 0)),
            pl.BlockSpec((BR, H), lambda i: (i, 0)),
        ],
        out_shape=[
            jax.ShapeDtypeStruct((N, H), jnp.float32),
            jax.ShapeDtypeStruct((N, H), jnp.float32),
        ],
    )(x, W0, b0.reshape(1, H), g0.reshape(1, H), beta0.reshape(1, H), degs)


def _tc_hop_body(p_ref, hs_ref, degd_ref, w_ref, b_ref, h_ref, degs_ref,
                 hout_ref, hsout_ref):
    dd = degd_ref[0, :, 0:1] + degd_ref[1, :, 0:1] + 1.0
    agg = (p_ref[0] + p_ref[1] + hs_ref[...]) * lax.rsqrt(dd)
    t = jnp.dot(agg, w_ref[...], preferred_element_type=jnp.float32)
    t = jnp.maximum(t + b_ref[...], 0.0)
    hn = t + h_ref[...]
    ds_ = degs_ref[0, :, 0:1] + degs_ref[1, :, 0:1] + 1.0
    hout_ref[...] = hn
    hsout_ref[...] = hn * lax.rsqrt(ds_)


def _tc_hop(p, hs, degd, W, b, h, degs):
    return pl.pallas_call(
        _tc_hop_body,
        grid=(N // BR,),
        in_specs=[
            pl.BlockSpec((NC, BR, H), lambda i: (0, i, 0)),
            pl.BlockSpec((BR, H), lambda i: (i, 0)),
            pl.BlockSpec((NC, BR, 16), lambda i: (0, i, 0)),
            pl.BlockSpec((H, H), lambda i: (0, 0)),
            pl.BlockSpec((1, H), lambda i: (0, 0)),
            pl.BlockSpec((BR, H), lambda i: (i, 0)),
            pl.BlockSpec((NC, BR, 16), lambda i: (0, i, 0)),
        ],
        out_specs=[
            pl.BlockSpec((BR, H), lambda i: (i, 0)),
            pl.BlockSpec((BR, H), lambda i: (i, 0)),
        ],
        out_shape=[
            jax.ShapeDtypeStruct((N, H), jnp.float32),
            jax.ShapeDtypeStruct((N, H), jnp.float32),
        ],
    )(p, hs, degd, W, b.reshape(1, H), h, degs)


def kernel(x, edge_index, W0, b0, g0, beta0, W1, b1, W2, b2, W3, b3):
    src = edge_index[0]
    dst = edge_index[1]
    zeros16 = jnp.zeros((NPAD, 16), jnp.float32)
    ones16 = jnp.ones((K, 16), jnp.float32)
    zeros_big = jnp.zeros((NPAD, H), jnp.float32)

    degs, degd = _sc_degree(src, dst, zeros16, ones16)
    h, hs = _tc_mlp(x, W0, b0, g0, beta0, degs)
    for (W, b) in ((W1, b1), (W2, b2), (W3, b3)):
        p = _sc_aggregate(hs, src, dst, zeros_big)
        h, hs = _tc_hop(p, hs, degd, W, b, h, degs)
    return h
